# baseline (speedup 1.0000x reference)
import jax
import jax.numpy as jnp
from jax import lax
from jax.experimental import pallas as pl

X, Y, DIM = 64, 64, 128
N = X * Y
NUM_ITERS = 100
LEARNING_RADIUS = 0.5
SIGMA = max(X, Y) / 2.0
COGNITIVE, SOCIAL, INERTIA = 0.01, 0.1, 0.001

BLK = 512
NBLK = N // BLK


def _som_block(iv_ref, p_full_ref, gl_f_ref, gl_t_ref, params_ref,
               gl_blk_ref, p_blk_ref, v_blk_ref, r1_ref, r2_ref,
               out_p_ref, out_v_ref):
    lr = params_ref[0:1, 0:1]
    s2 = params_ref[0:1, 1:2]

    diff = iv_ref[:] - p_full_ref[:] + 1e-6
    dists = jnp.sqrt(jnp.sum(diff * diff, axis=1, keepdims=True))
    dmin = jnp.min(dists)
    iota = lax.broadcasted_iota(jnp.int32, (N, 1), 0)
    bmu = jnp.min(jnp.where(dists <= dmin, iota, N))

    gl_row = gl_f_ref[pl.ds(bmu, 1), :]
    bx = gl_row[0:1, 0:1]
    by = gl_row[0:1, 1:2]

    d2_row = (gl_t_ref[0:1, :] - bx) ** 2 + (gl_t_ref[1:2, :] - by) ** 2
    nbhd_row = jnp.exp(-(d2_row / s2))
    dxy = gl_blk_ref[:, :] - gl_row
    d2_col = jnp.sum(dxy * dxy, axis=1, keepdims=True)
    nbhd_i = jnp.exp(-(d2_col / s2))

    thresh = nbhd_i + lr
    acc = jnp.zeros((BLK, DIM), jnp.float32)
    cnt = jnp.zeros((BLK, 1), jnp.float32)
    for j in range(NBLK):
        maskf = (nbhd_row[:, j * BLK:(j + 1) * BLK] <= thresh).astype(jnp.float32)
        acc = acc + jnp.dot(maskf, p_full_ref[j * BLK:(j + 1) * BLK, :],
                            preferred_element_type=jnp.float32)
        cnt = cnt + jnp.sum(maskf, axis=1, keepdims=True)
    centroid = acc / cnt

    global_best = p_full_ref[pl.ds(bmu, 1), :]
    upd = (1.0 - nbhd_i) <= lr

    p_blk = p_blk_ref[:, :]
    v_blk = v_blk_ref[:, :]
    v_cog = COGNITIVE * r1_ref[:, :] * (centroid - p_blk)
    v_soc = SOCIAL * r2_ref[:, :] * (global_best - p_blk)
    v_upd = INERTIA * v_blk + v_cog + v_soc
    out_v_ref[:, :] = jnp.where(upd, v_upd, v_blk)
    out_p_ref[:, :] = jnp.where(upd, p_blk + v_upd, p_blk)


def kernel(input_vec, iter_num, particles, velocities, grid_locations, r1, r2):
    decay = 1.0 - iter_num / NUM_ITERS
    lr_decay = LEARNING_RADIUS * decay
    sigma_decay = SIGMA * decay
    s2 = sigma_decay ** 2
    params = jnp.zeros((1, 128), jnp.float32)
    params = params.at[0, 0].set(lr_decay).at[0, 1].set(s2)

    gl_f = grid_locations.astype(jnp.float32)
    gl_t = gl_f.T
    iv = input_vec.reshape(1, DIM)

    full = lambda shape: pl.BlockSpec(shape, lambda i: (0, 0))
    blk = pl.BlockSpec((BLK, DIM), lambda i: (i, 0))

    out_p, out_v = pl.pallas_call(
        _som_block,
        grid=(NBLK,),
        in_specs=[
            full((1, DIM)),
            full((N, DIM)),
            full((N, 2)),
            full((2, N)),
            full((1, 128)),
            pl.BlockSpec((BLK, 2), lambda i: (i, 0)),
            blk,
            blk,
            blk,
            blk,
        ],
        out_specs=[blk, blk],
        out_shape=[
            jax.ShapeDtypeStruct((N, DIM), jnp.float32),
            jax.ShapeDtypeStruct((N, DIM), jnp.float32),
        ],
    )(iv, particles, gl_f, gl_t, params, gl_f, particles, velocities, r1, r2)
    return out_p, out_v
